# baseline (device time: 35660 ns/iter reference)
import jax
import jax.numpy as jnp
from jax import lax
from jax.experimental import pallas as pl
from jax.experimental.pallas import tpu as pltpu

N_DEV = 4
SQ = 256
SKV = 4096
H_LOC = 8
DH = 128
QB = 64
N_QB = SQ // QB
KV_PER_QB = SKV // N_QB
NT = KV_PER_QB // QB
D_MODEL = 1024
QCOL = D_MODEL // 4
SCALE = 0.08838834764831843


def kernel(x, Wq, K_ext, V_ext, Wo):
    x2d = x.reshape(SQ, D_MODEL)
    k5 = K_ext.reshape(NT, N_QB, QB, H_LOC, DH)
    v5 = V_ext.reshape(NT, N_QB, QB, H_LOC, DH)

    def body(x_ref, wq_hbm, k_hbm, v_hbm, wo_hbm, out_ref,
             wq_scr, k_scr, v_scr, wo_scr,
             wq_sem, wo_sem, k_sems, v_sems,
             snd, rcv, send_sems, recv_sems):
        my_pos = lax.axis_index("i")

        barrier_sem = pltpu.get_barrier_semaphore()
        for o in range(1, N_DEV):
            pl.semaphore_signal(
                barrier_sem, inc=1,
                device_id=(lax.rem(my_pos + o, N_DEV),),
                device_id_type=pl.DeviceIdType.MESH,
            )
        pl.semaphore_wait(barrier_sem, N_DEV - 1)

        wq_dma = pltpu.make_async_copy(
            wq_hbm.at[:, pl.ds(my_pos * H_LOC * DH, H_LOC * DH)],
            wq_scr, wq_sem)
        wq_dma.start()

        def issue_unit(u, slot):
            quarter, qb = divmod(u, N_QB)
            dmas = []
            for i in range(2):
                h = 2 * quarter + i
                dmas.append(pltpu.make_async_copy(
                    k_hbm.at[:, qb, :, h, :], k_scr.at[slot, i],
                    k_sems.at[slot]))
                dmas.append(pltpu.make_async_copy(
                    v_hbm.at[:, qb, :, h, :], v_scr.at[slot, i],
                    v_sems.at[slot]))
            for d in dmas:
                d.start()
            return dmas

        kv = {u: issue_unit(u, u % 6) for u in range(6)}
        wo_dma = pltpu.make_async_copy(wo_hbm, wo_scr, wo_sem)
        wo_dma.start()

        def mk_send(q, o):
            return pltpu.make_async_remote_copy(
                src_ref=snd.at[q],
                dst_ref=rcv.at[q, o - 1],
                send_sem=send_sems.at[q, o - 1],
                recv_sem=recv_sems.at[q, o - 1],
                device_id=(lax.rem(my_pos + o, N_DEV),),
                device_id_type=pl.DeviceIdType.MESH,
            )

        sends = [[mk_send(q, o) for o in range(1, N_DEV)] for q in range(4)]

        def drain(q):
            for s in range(N_DEV - 1):
                sends[q][s].wait_recv()
                org = lax.rem(my_pos - (s + 1) + N_DEV, N_DEV)
                out_ref[...] += jnp.dot(
                    rcv[q, s],
                    wo_scr[pl.ds(org * D_MODEL + q * QCOL, QCOL),
                           :].astype(jnp.bfloat16),
                    preferred_element_type=jnp.float32,
                )

        wq_dma.wait()
        q16 = (jnp.dot(x_ref[...].astype(jnp.bfloat16),
                       wq_scr[...].astype(jnp.bfloat16),
                       preferred_element_type=jnp.float32)
               * SCALE).astype(jnp.bfloat16)

        for quarter in range(4):
            rows = []
            for qb in range(N_QB):
                u = quarter * N_QB + qb
                slot = u % 6
                for d in kv[u]:
                    d.wait()
                k_u = k_scr[slot].astype(jnp.bfloat16).reshape(
                    2, KV_PER_QB, DH)
                v_u = v_scr[slot].astype(jnp.bfloat16).reshape(
                    2, KV_PER_QB, DH)
                pair_ctx = []
                for i in range(2):
                    h = 2 * quarter + i
                    q_h = q16[qb * QB:(qb + 1) * QB, h * DH:(h + 1) * DH]
                    s = lax.dot_general(
                        q_h, k_u[i], (((1,), (1,)), ((), ())),
                        preferred_element_type=jnp.float32)
                    w = jnp.exp(s.astype(jnp.bfloat16))
                    denom = jnp.sum(w.astype(jnp.float32), axis=1,
                                    keepdims=True)
                    ctx_h = jnp.dot(w, v_u[i],
                                    preferred_element_type=jnp.float32
                                    ) / denom
                    pair_ctx.append(ctx_h)
                if u + 6 < 16:
                    kv[u + 6] = issue_unit(u + 6, slot)
                rows.append(jnp.concatenate(pair_ctx, axis=1))
            ctx_q = jnp.concatenate(rows, axis=0).astype(jnp.bfloat16)
            snd[quarter] = ctx_q
            for s in sends[quarter]:
                s.start()

            if quarter == 0:
                wo_dma.wait()
            contrib = jnp.dot(
                ctx_q,
                wo_scr[pl.ds(my_pos * D_MODEL + quarter * QCOL, QCOL),
                       :].astype(jnp.bfloat16),
                preferred_element_type=jnp.float32,
            )
            if quarter == 0:
                out_ref[...] = contrib
            else:
                out_ref[...] += contrib
            if quarter >= 1:
                drain(quarter - 1)
        drain(3)

        for q in range(4):
            for s in sends[q]:
                s.wait_send()

    out2d = pl.pallas_call(
        body,
        out_shape=jax.ShapeDtypeStruct((SQ, D_MODEL), jnp.float32),
        in_specs=[
            pl.BlockSpec(memory_space=pltpu.VMEM),
            pl.BlockSpec(memory_space=pl.ANY),
            pl.BlockSpec(memory_space=pl.ANY),
            pl.BlockSpec(memory_space=pl.ANY),
            pl.BlockSpec(memory_space=pl.ANY),
        ],
        out_specs=pl.BlockSpec(memory_space=pltpu.VMEM),
        scratch_shapes=[
            pltpu.VMEM((D_MODEL, H_LOC * DH), jnp.float32),
            pltpu.VMEM((6, 2, NT, QB, DH), jnp.float32),
            pltpu.VMEM((6, 2, NT, QB, DH), jnp.float32),
            pltpu.VMEM((SKV, D_MODEL), jnp.float32),
            pltpu.SemaphoreType.DMA,
            pltpu.SemaphoreType.DMA,
            pltpu.SemaphoreType.DMA((6,)),
            pltpu.SemaphoreType.DMA((6,)),
            pltpu.VMEM((4, SQ, QCOL), jnp.bfloat16),
            pltpu.VMEM((4, 3, SQ, QCOL), jnp.bfloat16),
            pltpu.SemaphoreType.DMA((4, 3)),
            pltpu.SemaphoreType.DMA((4, 3)),
        ],
        compiler_params=pltpu.CompilerParams(
            collective_id=0, vmem_limit_bytes=60 * 1024 * 1024),
    )(x2d, Wq, k5, v5, Wo)

    return out2d.reshape(1, SQ, D_MODEL)


# device time: 35602 ns/iter; 1.0016x vs baseline; 1.0016x over previous
import jax
import jax.numpy as jnp
from jax import lax
from jax.experimental import pallas as pl
from jax.experimental.pallas import tpu as pltpu

N_DEV = 4
SQ = 256
SKV = 4096
H_LOC = 8
DH = 128
QB = 64
N_QB = SQ // QB
KV_PER_QB = SKV // N_QB
NT = KV_PER_QB // QB
D_MODEL = 1024
QCOL = D_MODEL // 4
SCALE = 0.08838834764831843


def kernel(x, Wq, K_ext, V_ext, Wo):
    x2d = x.reshape(SQ, D_MODEL)
    k5 = K_ext.reshape(NT, N_QB, QB, H_LOC, DH)
    v5 = V_ext.reshape(NT, N_QB, QB, H_LOC, DH)

    def body(x_ref, wq_hbm, k_hbm, v_hbm, wo_hbm, out_ref,
             wq_scr, k_scr, v_scr, wo_scr,
             wq_sem, wo_sem, k_sems, v_sems,
             snd, rcv, send_sems, recv_sems):
        my_pos = lax.axis_index("i")

        barrier_sem = pltpu.get_barrier_semaphore()
        for o in range(1, N_DEV):
            pl.semaphore_signal(
                barrier_sem, inc=1,
                device_id=(lax.rem(my_pos + o, N_DEV),),
                device_id_type=pl.DeviceIdType.MESH,
            )
        pl.semaphore_wait(barrier_sem, N_DEV - 1)

        wq_dma = pltpu.make_async_copy(
            wq_hbm.at[:, pl.ds(my_pos * H_LOC * DH, H_LOC * DH)],
            wq_scr, wq_sem)
        wq_dma.start()

        def issue_unit(u, slot):
            quarter, qb = divmod(u, N_QB)
            dmas = []
            for i in range(2):
                h = 2 * quarter + i
                dmas.append(pltpu.make_async_copy(
                    k_hbm.at[:, qb, :, h, :], k_scr.at[slot, i],
                    k_sems.at[slot]))
                dmas.append(pltpu.make_async_copy(
                    v_hbm.at[:, qb, :, h, :], v_scr.at[slot, i],
                    v_sems.at[slot]))
            for d in dmas:
                d.start()
            return dmas

        kv = {u: issue_unit(u, u % 6) for u in range(6)}
        wo_dma = pltpu.make_async_copy(wo_hbm, wo_scr, wo_sem)
        wo_dma.start()

        def mk_send(q, o):
            return pltpu.make_async_remote_copy(
                src_ref=snd.at[q],
                dst_ref=rcv.at[q, o - 1],
                send_sem=send_sems.at[q, o - 1],
                recv_sem=recv_sems.at[q, o - 1],
                device_id=(lax.rem(my_pos + o, N_DEV),),
                device_id_type=pl.DeviceIdType.MESH,
            )

        sends = [[mk_send(q, o) for o in range(1, N_DEV)] for q in range(4)]

        def drain(q):
            for s in range(N_DEV - 1):
                sends[q][s].wait_recv()
                org = lax.rem(my_pos - (s + 1) + N_DEV, N_DEV)
                out_ref[...] += jnp.dot(
                    rcv[q, s],
                    wo_scr[pl.ds(org * D_MODEL + q * QCOL, QCOL),
                           :].astype(jnp.bfloat16),
                    preferred_element_type=jnp.float32,
                )

        wq_dma.wait()
        q16 = (jnp.dot(x_ref[...].astype(jnp.bfloat16),
                       wq_scr[...].astype(jnp.bfloat16),
                       preferred_element_type=jnp.float32)
               * SCALE).astype(jnp.bfloat16)

        for quarter in range(4):
            rows = []
            for qb in range(N_QB):
                u = quarter * N_QB + qb
                slot = u % 6
                for d in kv[u]:
                    d.wait()
                k_u = k_scr[slot].astype(jnp.bfloat16).reshape(
                    2, KV_PER_QB, DH)
                v_u = v_scr[slot].astype(jnp.bfloat16).reshape(
                    2, KV_PER_QB, DH)
                pair_ctx = []
                for i in range(2):
                    h = 2 * quarter + i
                    q_h = q16[qb * QB:(qb + 1) * QB, h * DH:(h + 1) * DH]
                    s = lax.dot_general(
                        q_h, k_u[i], (((1,), (1,)), ((), ())),
                        preferred_element_type=jnp.float32)
                    w = jnp.exp(s)
                    denom = jnp.sum(w, axis=1, keepdims=True)
                    ctx_h = jnp.dot(w.astype(jnp.bfloat16), v_u[i],
                                    preferred_element_type=jnp.float32
                                    ) / denom
                    pair_ctx.append(ctx_h)
                if u + 6 < 16:
                    kv[u + 6] = issue_unit(u + 6, slot)
                rows.append(jnp.concatenate(pair_ctx, axis=1))
            ctx_q = jnp.concatenate(rows, axis=0).astype(jnp.bfloat16)
            snd[quarter] = ctx_q
            for s in sends[quarter]:
                s.start()

            if quarter == 0:
                wo_dma.wait()
            contrib = jnp.dot(
                ctx_q,
                wo_scr[pl.ds(my_pos * D_MODEL + quarter * QCOL, QCOL),
                       :].astype(jnp.bfloat16),
                preferred_element_type=jnp.float32,
            )
            if quarter == 0:
                out_ref[...] = contrib
            else:
                out_ref[...] += contrib
            if quarter >= 1:
                drain(quarter - 1)
        drain(3)

        for q in range(4):
            for s in sends[q]:
                s.wait_send()

    out2d = pl.pallas_call(
        body,
        out_shape=jax.ShapeDtypeStruct((SQ, D_MODEL), jnp.float32),
        in_specs=[
            pl.BlockSpec(memory_space=pltpu.VMEM),
            pl.BlockSpec(memory_space=pl.ANY),
            pl.BlockSpec(memory_space=pl.ANY),
            pl.BlockSpec(memory_space=pl.ANY),
            pl.BlockSpec(memory_space=pl.ANY),
        ],
        out_specs=pl.BlockSpec(memory_space=pltpu.VMEM),
        scratch_shapes=[
            pltpu.VMEM((D_MODEL, H_LOC * DH), jnp.float32),
            pltpu.VMEM((6, 2, NT, QB, DH), jnp.float32),
            pltpu.VMEM((6, 2, NT, QB, DH), jnp.float32),
            pltpu.VMEM((SKV, D_MODEL), jnp.float32),
            pltpu.SemaphoreType.DMA,
            pltpu.SemaphoreType.DMA,
            pltpu.SemaphoreType.DMA((6,)),
            pltpu.SemaphoreType.DMA((6,)),
            pltpu.VMEM((4, SQ, QCOL), jnp.bfloat16),
            pltpu.VMEM((4, 3, SQ, QCOL), jnp.bfloat16),
            pltpu.SemaphoreType.DMA((4, 3)),
            pltpu.SemaphoreType.DMA((4, 3)),
        ],
        compiler_params=pltpu.CompilerParams(
            collective_id=0, vmem_limit_bytes=60 * 1024 * 1024),
    )(x2d, Wq, k5, v5, Wo)

    return out2d.reshape(1, SQ, D_MODEL)


# device time: 34678 ns/iter; 1.0283x vs baseline; 1.0266x over previous
import jax
import jax.numpy as jnp
from jax import lax
from jax.experimental import pallas as pl
from jax.experimental.pallas import tpu as pltpu

N_DEV = 4
SQ = 256
SKV = 4096
H_LOC = 8
DH = 128
QB = 64
N_QB = SQ // QB
KV_PER_QB = SKV // N_QB
NT = KV_PER_QB // QB
D_MODEL = 1024
QCOL = D_MODEL // 4
SCALE = 0.08838834764831843


def kernel(x, Wq, K_ext, V_ext, Wo):
    x2d = x.reshape(SQ, D_MODEL)
    k5 = K_ext.reshape(NT, N_QB, QB, H_LOC, DH)
    v5 = V_ext.reshape(NT, N_QB, QB, H_LOC, DH)

    def body(x_ref, wq_hbm, k_hbm, v_hbm, wo_hbm, out_ref,
             wq_scr, k_scr, v_scr, wo_scr,
             wq_sem, wo_sem, k_sems, v_sems,
             snd, rcv, send_sems, recv_sems):
        my_pos = lax.axis_index("i")

        barrier_sem = pltpu.get_barrier_semaphore()
        for o in range(1, N_DEV):
            pl.semaphore_signal(
                barrier_sem, inc=1,
                device_id=(lax.rem(my_pos + o, N_DEV),),
                device_id_type=pl.DeviceIdType.MESH,
            )
        pl.semaphore_wait(barrier_sem, N_DEV - 1)

        wq_dma = pltpu.make_async_copy(
            wq_hbm.at[:, pl.ds(my_pos * H_LOC * DH, H_LOC * DH)],
            wq_scr, wq_sem)
        wq_dma.start()

        def issue_unit(u, slot):
            quarter, qb = divmod(u, N_QB)
            dmas = []
            for i in range(2):
                h = 2 * quarter + i
                dmas.append(pltpu.make_async_copy(
                    k_hbm.at[:, qb, :, h, :], k_scr.at[slot, i],
                    k_sems.at[slot]))
                dmas.append(pltpu.make_async_copy(
                    v_hbm.at[:, qb, :, h, :], v_scr.at[slot, i],
                    v_sems.at[slot]))
            for d in dmas:
                d.start()
            return dmas

        kv = {u: issue_unit(u, u % 4) for u in range(4)}
        wo_dma = pltpu.make_async_copy(wo_hbm, wo_scr, wo_sem)
        wo_dma.start()

        def mk_send(q, o):
            return pltpu.make_async_remote_copy(
                src_ref=snd.at[q],
                dst_ref=rcv.at[q, o - 1],
                send_sem=send_sems.at[q, o - 1],
                recv_sem=recv_sems.at[q, o - 1],
                device_id=(lax.rem(my_pos + o, N_DEV),),
                device_id_type=pl.DeviceIdType.MESH,
            )

        sends = [[mk_send(q, o) for o in range(1, N_DEV)] for q in range(4)]

        def drain(q):
            for s in range(N_DEV - 1):
                sends[q][s].wait_recv()
                org = lax.rem(my_pos - (s + 1) + N_DEV, N_DEV)
                out_ref[...] += jnp.dot(
                    rcv[q, s],
                    wo_scr[pl.ds(org * D_MODEL + q * QCOL, QCOL),
                           :].astype(jnp.bfloat16),
                    preferred_element_type=jnp.float32,
                )

        wq_dma.wait()
        q16 = (jnp.dot(x_ref[...].astype(jnp.bfloat16),
                       wq_scr[...].astype(jnp.bfloat16),
                       preferred_element_type=jnp.float32)
               * SCALE).astype(jnp.bfloat16)

        for quarter in range(4):
            rows = []
            for qb in range(N_QB):
                u = quarter * N_QB + qb
                slot = u % 4
                for d in kv[u]:
                    d.wait()
                k_u = k_scr[slot].astype(jnp.bfloat16).reshape(
                    2, KV_PER_QB, DH)
                v_u = v_scr[slot].astype(jnp.bfloat16).reshape(
                    2, KV_PER_QB, DH)
                pair_ctx = []
                for i in range(2):
                    h = 2 * quarter + i
                    q_h = q16[qb * QB:(qb + 1) * QB, h * DH:(h + 1) * DH]
                    s = lax.dot_general(
                        q_h, k_u[i], (((1,), (1,)), ((), ())),
                        preferred_element_type=jnp.float32)
                    w = jnp.exp(s)
                    denom = jnp.sum(w, axis=1, keepdims=True)
                    ctx_h = jnp.dot(w.astype(jnp.bfloat16), v_u[i],
                                    preferred_element_type=jnp.float32
                                    ) / denom
                    pair_ctx.append(ctx_h)
                if u + 4 < 16:
                    kv[u + 4] = issue_unit(u + 4, slot)
                rows.append(jnp.concatenate(pair_ctx, axis=1))
            ctx_q = jnp.concatenate(rows, axis=0).astype(jnp.bfloat16)
            snd[quarter] = ctx_q
            for s in sends[quarter]:
                s.start()

            if quarter == 0:
                wo_dma.wait()
            contrib = jnp.dot(
                ctx_q,
                wo_scr[pl.ds(my_pos * D_MODEL + quarter * QCOL, QCOL),
                       :].astype(jnp.bfloat16),
                preferred_element_type=jnp.float32,
            )
            if quarter == 0:
                out_ref[...] = contrib
            else:
                out_ref[...] += contrib
            if quarter >= 1:
                drain(quarter - 1)
        drain(3)

        for q in range(4):
            for s in sends[q]:
                s.wait_send()

    out2d = pl.pallas_call(
        body,
        out_shape=jax.ShapeDtypeStruct((SQ, D_MODEL), jnp.float32),
        in_specs=[
            pl.BlockSpec(memory_space=pltpu.VMEM),
            pl.BlockSpec(memory_space=pl.ANY),
            pl.BlockSpec(memory_space=pl.ANY),
            pl.BlockSpec(memory_space=pl.ANY),
            pl.BlockSpec(memory_space=pl.ANY),
        ],
        out_specs=pl.BlockSpec(memory_space=pltpu.VMEM),
        scratch_shapes=[
            pltpu.VMEM((D_MODEL, H_LOC * DH), jnp.float32),
            pltpu.VMEM((4, 2, NT, QB, DH), jnp.float32),
            pltpu.VMEM((4, 2, NT, QB, DH), jnp.float32),
            pltpu.VMEM((SKV, D_MODEL), jnp.float32),
            pltpu.SemaphoreType.DMA,
            pltpu.SemaphoreType.DMA,
            pltpu.SemaphoreType.DMA((4,)),
            pltpu.SemaphoreType.DMA((4,)),
            pltpu.VMEM((4, SQ, QCOL), jnp.bfloat16),
            pltpu.VMEM((4, 3, SQ, QCOL), jnp.bfloat16),
            pltpu.SemaphoreType.DMA((4, 3)),
            pltpu.SemaphoreType.DMA((4, 3)),
        ],
        compiler_params=pltpu.CompilerParams(
            collective_id=0, vmem_limit_bytes=60 * 1024 * 1024),
    )(x2d, Wq, k5, v5, Wo)

    return out2d.reshape(1, SQ, D_MODEL)


# device time: 30745 ns/iter; 1.1599x vs baseline; 1.1279x over previous
import jax
import jax.numpy as jnp
from jax import lax
from jax.experimental import pallas as pl
from jax.experimental.pallas import tpu as pltpu

N_DEV = 4
SQ = 256
SKV = 4096
H_LOC = 8
DH = 128
QB = 64
N_QB = SQ // QB
KV_PER_QB = SKV // N_QB
NT = KV_PER_QB // QB
D_MODEL = 1024
QCOL = D_MODEL // 4
SCALE = 0.08838834764831843


def kernel(x, Wq, K_ext, V_ext, Wo):
    x2d = x.reshape(SQ, D_MODEL)
    k5 = K_ext.reshape(NT, N_QB, QB, H_LOC, DH)
    v5 = V_ext.reshape(NT, N_QB, QB, H_LOC, DH)
    wo4 = Wo.reshape(N_DEV, 4, QCOL, D_MODEL)

    def body(x_ref, wq_hbm, k_hbm, v_hbm, wo_hbm, out_ref,
             wq_scr, k_scr, v_scr, wo_slc,
             wq_sem, k_sems, v_sems, wo_sems,
             snd, rcv, send_sems, recv_sems):
        my_pos = lax.axis_index("i")

        barrier_sem = pltpu.get_barrier_semaphore()
        for o in range(1, N_DEV):
            pl.semaphore_signal(
                barrier_sem, inc=1,
                device_id=(lax.rem(my_pos + o, N_DEV),),
                device_id_type=pl.DeviceIdType.MESH,
            )
        pl.semaphore_wait(barrier_sem, N_DEV - 1)

        wq_dma = pltpu.make_async_copy(
            wq_hbm.at[:, pl.ds(my_pos * H_LOC * DH, H_LOC * DH)],
            wq_scr, wq_sem)
        wq_dma.start()

        def issue_unit(u, slot):
            quarter, qb = divmod(u, N_QB)
            dmas = []
            for i in range(2):
                h = 2 * quarter + i
                dmas.append(pltpu.make_async_copy(
                    k_hbm.at[:, qb, :, h, :], k_scr.at[slot, i],
                    k_sems.at[slot]))
                dmas.append(pltpu.make_async_copy(
                    v_hbm.at[:, qb, :, h, :], v_scr.at[slot, i],
                    v_sems.at[slot]))
            for d in dmas:
                d.start()
            return dmas

        kv = {u: issue_unit(u, u % 4) for u in range(4)}

        m1 = lax.rem(my_pos - 1 + N_DEV, N_DEV)
        m2 = lax.rem(my_pos - 2 + N_DEV, N_DEV)
        m3 = lax.rem(my_pos - 3 + N_DEV, N_DEV)
        reqs = [(my_pos, 0), (my_pos, 1),
                (m1, 0), (m2, 0), (m3, 0), (my_pos, 2),
                (m1, 1), (m2, 1), (m3, 1), (my_pos, 3),
                (m1, 2), (m2, 2), (m3, 2),
                (m1, 3), (m2, 3), (m3, 3)]
        wo_dmas = {}
        wo_state = {"i": 0}

        def wo_issue(j):
            org, qq = reqs[j]
            d = pltpu.make_async_copy(
                wo_hbm.at[org, qq], wo_slc.at[j % 4], wo_sems.at[j % 4])
            d.start()
            wo_dmas[j] = d

        for j in range(4):
            wo_issue(j)

        def wo_next():
            j = wo_state["i"]
            wo_state["i"] += 1
            wo_dmas[j].wait()
            if j + 4 < 16:
                wo_issue(j + 4)
            return wo_slc[j % 4].astype(jnp.bfloat16)

        def mk_send(q, o):
            return pltpu.make_async_remote_copy(
                src_ref=snd.at[q],
                dst_ref=rcv.at[q, o - 1],
                send_sem=send_sems.at[q, o - 1],
                recv_sem=recv_sems.at[q, o - 1],
                device_id=(lax.rem(my_pos + o, N_DEV),),
                device_id_type=pl.DeviceIdType.MESH,
            )

        sends = [[mk_send(q, o) for o in range(1, N_DEV)] for q in range(4)]

        def drain(q):
            for s in range(N_DEV - 1):
                sends[q][s].wait_recv()
                out_ref[...] += jnp.dot(
                    rcv[q, s], wo_next(),
                    preferred_element_type=jnp.float32,
                )

        wq_dma.wait()
        q16 = (jnp.dot(x_ref[...].astype(jnp.bfloat16),
                       wq_scr[...].astype(jnp.bfloat16),
                       preferred_element_type=jnp.float32)
               * SCALE).astype(jnp.bfloat16)

        for quarter in range(4):
            rows = []
            for qb in range(N_QB):
                u = quarter * N_QB + qb
                slot = u % 4
                for d in kv[u]:
                    d.wait()
                k_u = k_scr[slot].astype(jnp.bfloat16).reshape(
                    2, KV_PER_QB, DH)
                v_u = v_scr[slot].astype(jnp.bfloat16).reshape(
                    2, KV_PER_QB, DH)
                pair_ctx = []
                for i in range(2):
                    h = 2 * quarter + i
                    q_h = q16[qb * QB:(qb + 1) * QB, h * DH:(h + 1) * DH]
                    s = lax.dot_general(
                        q_h, k_u[i], (((1,), (1,)), ((), ())),
                        preferred_element_type=jnp.float32)
                    w = jnp.exp(s)
                    denom = jnp.sum(w, axis=1, keepdims=True)
                    ctx_h = jnp.dot(w.astype(jnp.bfloat16), v_u[i],
                                    preferred_element_type=jnp.float32
                                    ) / denom
                    pair_ctx.append(ctx_h)
                if u + 4 < 16:
                    kv[u + 4] = issue_unit(u + 4, slot)
                rows.append(jnp.concatenate(pair_ctx, axis=1))
            ctx_q = jnp.concatenate(rows, axis=0).astype(jnp.bfloat16)
            snd[quarter] = ctx_q
            for s in sends[quarter]:
                s.start()

            contrib = jnp.dot(ctx_q, wo_next(),
                              preferred_element_type=jnp.float32)
            if quarter == 0:
                out_ref[...] = contrib
            else:
                out_ref[...] += contrib
            if quarter >= 1:
                drain(quarter - 1)
        drain(3)

        for q in range(4):
            for s in sends[q]:
                s.wait_send()

    out2d = pl.pallas_call(
        body,
        out_shape=jax.ShapeDtypeStruct((SQ, D_MODEL), jnp.float32),
        in_specs=[
            pl.BlockSpec(memory_space=pltpu.VMEM),
            pl.BlockSpec(memory_space=pl.ANY),
            pl.BlockSpec(memory_space=pl.ANY),
            pl.BlockSpec(memory_space=pl.ANY),
            pl.BlockSpec(memory_space=pl.ANY),
        ],
        out_specs=pl.BlockSpec(memory_space=pltpu.VMEM),
        scratch_shapes=[
            pltpu.VMEM((D_MODEL, H_LOC * DH), jnp.float32),
            pltpu.VMEM((4, 2, NT, QB, DH), jnp.float32),
            pltpu.VMEM((4, 2, NT, QB, DH), jnp.float32),
            pltpu.VMEM((4, QCOL, D_MODEL), jnp.float32),
            pltpu.SemaphoreType.DMA,
            pltpu.SemaphoreType.DMA((4,)),
            pltpu.SemaphoreType.DMA((4,)),
            pltpu.SemaphoreType.DMA((4,)),
            pltpu.VMEM((4, SQ, QCOL), jnp.bfloat16),
            pltpu.VMEM((4, 3, SQ, QCOL), jnp.bfloat16),
            pltpu.SemaphoreType.DMA((4, 3)),
            pltpu.SemaphoreType.DMA((4, 3)),
        ],
        compiler_params=pltpu.CompilerParams(
            collective_id=0, vmem_limit_bytes=60 * 1024 * 1024),
    )(x2d, Wq, k5, v5, wo4)

    return out2d.reshape(1, SQ, D_MODEL)


# device time: 30520 ns/iter; 1.1684x vs baseline; 1.0074x over previous
import jax
import jax.numpy as jnp
from jax import lax
from jax.experimental import pallas as pl
from jax.experimental.pallas import tpu as pltpu

N_DEV = 4
SQ = 256
SKV = 4096
H_LOC = 8
DH = 128
QB = 64
N_QB = SQ // QB
KV_PER_QB = SKV // N_QB
NT = KV_PER_QB // QB
D_MODEL = 1024
QCOL = D_MODEL // 4
SCALE = 0.08838834764831843


def kernel(x, Wq, K_ext, V_ext, Wo):
    x2d = x.reshape(SQ, D_MODEL)
    k5 = K_ext.reshape(NT, N_QB, QB, H_LOC, DH)
    v5 = V_ext.reshape(NT, N_QB, QB, H_LOC, DH)
    wo4 = Wo.reshape(N_DEV, 4, QCOL, D_MODEL)

    def body(x_ref, wq_hbm, k_hbm, v_hbm, wo_hbm, out_ref,
             wq_scr, k_scr, v_scr, wo_slc,
             wq_sem, k_sems, v_sems, wo_sems,
             snd, rcv, send_sems, recv_sems):
        my_pos = lax.axis_index("i")

        barrier_sem = pltpu.get_barrier_semaphore()
        for o in range(1, N_DEV):
            pl.semaphore_signal(
                barrier_sem, inc=1,
                device_id=(lax.rem(my_pos + o, N_DEV),),
                device_id_type=pl.DeviceIdType.MESH,
            )
        pl.semaphore_wait(barrier_sem, N_DEV - 1)

        wq_dma = pltpu.make_async_copy(
            wq_hbm.at[:, pl.ds(my_pos * H_LOC * DH, H_LOC * DH)],
            wq_scr, wq_sem)
        wq_dma.start()

        def issue_unit(u, slot):
            quarter, qb = divmod(u, N_QB)
            dmas = []
            for i in range(2):
                h = 2 * quarter + i
                dmas.append(pltpu.make_async_copy(
                    k_hbm.at[:, qb, :, h, :], k_scr.at[slot, i],
                    k_sems.at[slot, i]))
                dmas.append(pltpu.make_async_copy(
                    v_hbm.at[:, qb, :, h, :], v_scr.at[slot, i],
                    v_sems.at[slot, i]))
            for d in dmas:
                d.start()
            return dmas

        kv = {u: issue_unit(u, u % 4) for u in range(4)}

        m1 = lax.rem(my_pos - 1 + N_DEV, N_DEV)
        m2 = lax.rem(my_pos - 2 + N_DEV, N_DEV)
        m3 = lax.rem(my_pos - 3 + N_DEV, N_DEV)
        reqs = [(my_pos, 0), (my_pos, 1),
                (m1, 0), (m2, 0), (m3, 0), (my_pos, 2),
                (m1, 1), (m2, 1), (m3, 1), (my_pos, 3),
                (m1, 2), (m2, 2), (m3, 2),
                (m1, 3), (m2, 3), (m3, 3)]
        wo_dmas = {}
        wo_state = {"i": 0}

        def wo_issue(j):
            org, qq = reqs[j]
            d = pltpu.make_async_copy(
                wo_hbm.at[org, qq], wo_slc.at[j % 4], wo_sems.at[j % 4])
            d.start()
            wo_dmas[j] = d

        for j in range(4):
            wo_issue(j)

        def wo_next():
            j = wo_state["i"]
            wo_state["i"] += 1
            wo_dmas[j].wait()
            if j + 4 < 16:
                wo_issue(j + 4)
            return wo_slc[j % 4].astype(jnp.bfloat16)

        def mk_send(q, o):
            return pltpu.make_async_remote_copy(
                src_ref=snd.at[q],
                dst_ref=rcv.at[q, o - 1],
                send_sem=send_sems.at[q, o - 1],
                recv_sem=recv_sems.at[q, o - 1],
                device_id=(lax.rem(my_pos + o, N_DEV),),
                device_id_type=pl.DeviceIdType.MESH,
            )

        sends = [[mk_send(q, o) for o in range(1, N_DEV)] for q in range(4)]

        def drain(q):
            for s in range(N_DEV - 1):
                sends[q][s].wait_recv()
                out_ref[...] += jnp.dot(
                    rcv[q, s], wo_next(),
                    preferred_element_type=jnp.float32,
                )

        wq_dma.wait()
        q16 = (jnp.dot(x_ref[...].astype(jnp.bfloat16),
                       wq_scr[...].astype(jnp.bfloat16),
                       preferred_element_type=jnp.float32)
               * SCALE).astype(jnp.bfloat16)

        for quarter in range(4):
            rows = []
            for qb in range(N_QB):
                u = quarter * N_QB + qb
                slot = u % 4
                for d in kv[u]:
                    d.wait()
                k_u = k_scr[slot].astype(jnp.bfloat16).reshape(
                    2, KV_PER_QB, DH)
                v_u = v_scr[slot].astype(jnp.bfloat16).reshape(
                    2, KV_PER_QB, DH)
                pair_ctx = []
                for i in range(2):
                    h = 2 * quarter + i
                    q_h = q16[qb * QB:(qb + 1) * QB, h * DH:(h + 1) * DH]
                    s = lax.dot_general(
                        q_h, k_u[i], (((1,), (1,)), ((), ())),
                        preferred_element_type=jnp.float32)
                    w = jnp.exp(s)
                    denom = jnp.sum(w, axis=1, keepdims=True)
                    ctx_h = jnp.dot(w.astype(jnp.bfloat16), v_u[i],
                                    preferred_element_type=jnp.float32
                                    ) / denom
                    pair_ctx.append(ctx_h)
                if u + 4 < 16:
                    kv[u + 4] = issue_unit(u + 4, slot)
                rows.append(jnp.concatenate(pair_ctx, axis=1))
            ctx_q = jnp.concatenate(rows, axis=0).astype(jnp.bfloat16)
            snd[quarter] = ctx_q
            for s in sends[quarter]:
                s.start()

            contrib = jnp.dot(ctx_q, wo_next(),
                              preferred_element_type=jnp.float32)
            if quarter == 0:
                out_ref[...] = contrib
            else:
                out_ref[...] += contrib
            if quarter >= 1:
                drain(quarter - 1)
        drain(3)

        for q in range(4):
            for s in sends[q]:
                s.wait_send()

    out2d = pl.pallas_call(
        body,
        out_shape=jax.ShapeDtypeStruct((SQ, D_MODEL), jnp.float32),
        in_specs=[
            pl.BlockSpec(memory_space=pltpu.VMEM),
            pl.BlockSpec(memory_space=pl.ANY),
            pl.BlockSpec(memory_space=pl.ANY),
            pl.BlockSpec(memory_space=pl.ANY),
            pl.BlockSpec(memory_space=pl.ANY),
        ],
        out_specs=pl.BlockSpec(memory_space=pltpu.VMEM),
        scratch_shapes=[
            pltpu.VMEM((D_MODEL, H_LOC * DH), jnp.float32),
            pltpu.VMEM((4, 2, NT, QB, DH), jnp.float32),
            pltpu.VMEM((4, 2, NT, QB, DH), jnp.float32),
            pltpu.VMEM((4, QCOL, D_MODEL), jnp.float32),
            pltpu.SemaphoreType.DMA,
            pltpu.SemaphoreType.DMA((4, 2)),
            pltpu.SemaphoreType.DMA((4, 2)),
            pltpu.SemaphoreType.DMA((4,)),
            pltpu.VMEM((4, SQ, QCOL), jnp.bfloat16),
            pltpu.VMEM((4, 3, SQ, QCOL), jnp.bfloat16),
            pltpu.SemaphoreType.DMA((4, 3)),
            pltpu.SemaphoreType.DMA((4, 3)),
        ],
        compiler_params=pltpu.CompilerParams(
            collective_id=0, vmem_limit_bytes=60 * 1024 * 1024),
    )(x2d, Wq, k5, v5, wo4)

    return out2d.reshape(1, SQ, D_MODEL)
